# Initial kernel scaffold; baseline (speedup 1.0000x reference)
import jax
import jax.numpy as jnp
from jax.experimental import pallas as pl


def kernel(que, img, emb, Wih_f, Whh_f, bih_f, bhh_f, Wih_b, Whh_b, bih_b, bhh_b, Wc, bc, g0, bt0, W11, b11, W12, b12, g1, bt1, W21, b21, W22, b22, g2, bt2):
    raise NotImplementedError("write your pallas kernel here")


# speedup vs baseline: 2.7314x; 2.7314x over previous
import jax
import jax.numpy as jnp
import numpy as np
from jax import lax
from jax.experimental import pallas as pl
from jax.experimental.pallas import tpu as pltpu
from jax.experimental.pallas import tpu_sc as plsc

FH = 14
FW = 14
D_WORD = 300
D_HID = 1024
D_EMB = 2048
SENT_LEN = 45
VOCAB = 10000
B = 32

_KP = 384
_NTOK = SENT_LEN * B
_NTOK_PAD = 1536
_ROWS_PER_W = 48
_FR = 256
_NP = B * _FR
_BLK = 1024
_NBLK = _NP // _BLK
_NVALID = float(B * FH * FW)
_F32 = jnp.float32


def _sc_gather(table, idx):
    mesh = plsc.VectorSubcoreMesh(core_axis_name="c", subcore_axis_name="s")

    def body(table_hbm, idx_hbm, out_hbm, idx_v, rows_v, sem):
        wid = lax.axis_index("s") * 2 + lax.axis_index("c")
        base = wid * _ROWS_PER_W
        pltpu.sync_copy(idx_hbm.at[pl.ds(base, _ROWS_PER_W)], idx_v)
        pltpu.async_copy(table_hbm.at[idx_v], rows_v, sem).wait()
        pltpu.sync_copy(rows_v, out_hbm.at[pl.ds(base, _ROWS_PER_W)])

    f = pl.kernel(
        body,
        mesh=mesh,
        out_type=jax.ShapeDtypeStruct((_NTOK_PAD, _KP), jnp.float32),
        scratch_types=[
            pltpu.VMEM((_ROWS_PER_W,), jnp.int32),
            pltpu.VMEM((_ROWS_PER_W, _KP), jnp.float32),
            pltpu.SemaphoreType.DMA,
        ],
    )
    return f(table, idx)


def _proj_body(x_ref, w_ref, o_ref):
    o_ref[...] = jnp.dot(x_ref[...], w_ref[...], preferred_element_type=_F32)


def _proj_call(x, w, mblk):
    m, k = x.shape
    n = w.shape[1]
    return pl.pallas_call(
        _proj_body,
        grid=(m // mblk,),
        in_specs=[pl.BlockSpec((mblk, k), lambda i: (i, 0)),
                  pl.BlockSpec((k, n), lambda i: (0, 0))],
        out_specs=pl.BlockSpec((mblk, n), lambda i: (i, 0)),
        out_shape=jax.ShapeDtypeStruct((m, n), jnp.float32),
    )(x, w)


def _lstm_body(xw_ref, whhf_ref, gb_ref, enc_ref):
    def step(t, hc):
        h, c = hc
        g = xw_ref[pl.ds(pl.multiple_of(t * B, B), B), :] + jnp.dot(
            h, whhf_ref[...], preferred_element_type=_F32)
        i = jax.nn.sigmoid(g[:, 0:1024])
        f = jax.nn.sigmoid(g[:, 1024:2048])
        gg = jnp.tanh(g[:, 2048:3072])
        o = jax.nn.sigmoid(g[:, 3072:4096])
        c = f * c + i * gg
        return (o * jnp.tanh(c), c)

    h0 = jnp.zeros((B, D_HID), _F32)
    h, _ = lax.fori_loop(0, SENT_LEN, step, (h0, h0))

    gb = gb_ref[...]
    cb = jax.nn.sigmoid(gb[:, 0:1024]) * jnp.tanh(gb[:, 2048:3072])
    hb = jax.nn.sigmoid(gb[:, 3072:4096]) * jnp.tanh(cb)

    e = jnp.concatenate([h, hb], axis=1)
    nrm = jnp.sqrt(jnp.sum(e * e, axis=1, keepdims=True))
    enc_ref[...] = e / jnp.maximum(nrm, 1e-12)


def _lstm_call(x, wihf, whhf, wihb):
    xw = _proj_call(x, wihf, 256)
    gb = _proj_call(x[(SENT_LEN - 1) * B:SENT_LEN * B, :], wihb, B)
    return pl.pallas_call(
        _lstm_body,
        out_shape=jax.ShapeDtypeStruct((B, D_EMB), jnp.float32),
    )(xw, whhf, gb)


def _norm_body(xp_ref, xn_ref):
    xs = xp_ref[...]
    ssq = jnp.sum(xs * xs, axis=1, keepdims=True)
    xn_ref[...] = xs / jnp.maximum(jnp.sqrt(ssq), 1e-12)


def _norm_call(xp):
    return pl.pallas_call(
        _norm_body,
        grid=(16,),
        in_specs=[pl.BlockSpec((512, 1024), lambda i: (i, 0))],
        out_specs=pl.BlockSpec((512, 1024), lambda i: (i, 0)),
        out_shape=jax.ShapeDtypeStruct((_NP, 1024), jnp.float32),
    )(xp)


def _tapsum(zb):
    acc = None
    for t in range(9):
        off = 16 * (t // 3 - 1) + (t % 3 - 1)
        r = pltpu.roll(zb[:, t * 128:(t + 1) * 128], (-off) % _BLK, 0)
        acc = r if acc is None else acc + r
    return acc


def _accum_sums(i, y, s_ref, s2_ref):
    ps = jnp.broadcast_to(jnp.sum(y, axis=0, keepdims=True), (8, 128))
    ps2 = jnp.broadcast_to(jnp.sum(y * y, axis=0, keepdims=True), (8, 128))

    @pl.when(i == 0)
    def _():
        s_ref[...] = ps
        s2_ref[...] = ps2

    @pl.when(i != 0)
    def _():
        s_ref[...] += ps
        s2_ref[...] += ps2


def _bn_from_sums(x, s_ref, s2_ref, g_ref, b_ref):
    m = s_ref[pl.ds(0, 1), :] / _NVALID
    m2 = s2_ref[pl.ds(0, 1), :] / _NVALID
    var = m2 - m * m
    xn = (x - m) / jnp.sqrt(var + 1e-5)
    return jnp.maximum(xn * g_ref[pl.ds(0, 1), :] + b_ref[pl.ds(0, 1), :],
                       0.0)


def _tap1_body(z_ref, pfr_ref, wcc_ref, mask_ref, yraw_ref, s_ref, s2_ref):
    i = pl.program_id(0)
    cm = jnp.dot(pfr_ref[...], wcc_ref[...], preferred_element_type=_F32)
    acc = _tapsum(z_ref[...]) + jnp.concatenate([cm, cm, cm, cm], axis=0)
    y = acc * mask_ref[...]
    yraw_ref[...] = y
    _accum_sums(i, y, s_ref, s2_ref)


def _tap1_call(z, pfr, wcc, mask):
    return pl.pallas_call(
        _tap1_body,
        grid=(_NBLK,),
        in_specs=[pl.BlockSpec((_BLK, 9 * 128), lambda i: (i, 0)),
                  pl.BlockSpec((_FR, 24), lambda i: (0, 0)),
                  pl.BlockSpec((24, 128), lambda i: (0, 0)),
                  pl.BlockSpec((_BLK, 128), lambda i: (0, 0))],
        out_specs=[pl.BlockSpec((_BLK, 128), lambda i: (i, 0)),
                   pl.BlockSpec((8, 128), lambda i: (0, 0)),
                   pl.BlockSpec((8, 128), lambda i: (0, 0))],
        out_shape=[jax.ShapeDtypeStruct((_NP, 128), jnp.float32),
                   jax.ShapeDtypeStruct((8, 128), jnp.float32),
                   jax.ShapeDtypeStruct((8, 128), jnp.float32)],
    )(z, pfr, wcc, mask)


def _mk_res_front(use_res):
    def body(raw_ref, s_ref, s2_ref, g_ref, b_ref, cba_ref, w1c_ref, w1v_ref,
             w2_ref, mask_ref, *rest):
        if use_res:
            res_ref = rest[0]
            v1o_ref, rawo_ref, so_ref, s2o_ref = rest[1:]
        else:
            v1o_ref, rawo_ref, so_ref, s2o_ref = rest
        i = pl.program_id(0)
        mask = mask_ref[...]
        vt = _bn_from_sums(raw_ref[...], s_ref, s2_ref, g_ref, b_ref) * mask
        if use_res:
            vt = vt + res_ref[...]
        cm1 = jnp.dot(cba_ref[...], w1c_ref[...], preferred_element_type=_F32)
        v1 = jnp.maximum(
            jnp.dot(vt, w1v_ref[...], preferred_element_type=_F32)
            + jnp.concatenate([cm1, cm1, cm1, cm1], axis=0), 0.0)
        v1o_ref[...] = v1
        z2 = jnp.dot(v1, w2_ref[...], preferred_element_type=_F32)
        y = _tapsum(z2) * mask
        rawo_ref[...] = y
        _accum_sums(i, y, so_ref, s2o_ref)

    def call(raw, s, s2, g, b, cba, w1c, w1v, w2all, mask, res=None):
        small = pl.BlockSpec((8, 128), lambda i: (0, 0))
        blk = pl.BlockSpec((_BLK, 128), lambda i: (i, 0))
        in_specs = [blk, small, small, small, small,
                    pl.BlockSpec((_FR, 8), lambda i: (0, 0)),
                    small,
                    pl.BlockSpec((128, 128), lambda i: (0, 0)),
                    pl.BlockSpec((128, 9 * 128), lambda i: (0, 0)),
                    pl.BlockSpec((_BLK, 128), lambda i: (0, 0))]
        args = [raw, s, s2, g, b, cba, w1c, w1v, w2all, mask]
        if use_res:
            in_specs.append(blk)
            args.append(res)
        return pl.pallas_call(
            body,
            grid=(_NBLK,),
            in_specs=in_specs,
            out_specs=[blk, blk, small, small],
            out_shape=[jax.ShapeDtypeStruct((_NP, 128), jnp.float32),
                       jax.ShapeDtypeStruct((_NP, 128), jnp.float32),
                       jax.ShapeDtypeStruct((8, 128), jnp.float32),
                       jax.ShapeDtypeStruct((8, 128), jnp.float32)],
        )(*args)

    return call


_res_front = _mk_res_front(False)
_res_front_r = _mk_res_front(True)


def _final_body(raw_ref, s_ref, s2_ref, g_ref, b_ref, mask_ref, res_ref,
                out_ref):
    y = _bn_from_sums(raw_ref[...], s_ref, s2_ref, g_ref, b_ref)
    out_ref[...] = y * mask_ref[...] + res_ref[...]


def _final_call(raw, s, s2, g, b, mask, res):
    small = pl.BlockSpec((8, 128), lambda i: (0, 0))
    blk = pl.BlockSpec((_BLK, 128), lambda i: (i, 0))
    return pl.pallas_call(
        _final_body,
        grid=(_NBLK,),
        in_specs=[blk, small, small, small, small,
                  pl.BlockSpec((_BLK, 128), lambda i: (0, 0)), blk],
        out_specs=blk,
        out_shape=jax.ShapeDtypeStruct((_NP, 128), jnp.float32),
    )(raw, s, s2, g, b, mask, res)


def _coord_consts():
    ii = np.arange(FH * FW)
    c0 = (ii / FW - FH // 2) / (FH / 2.0)
    c1 = (ii % FW - FW // 2) / (FW / 2.0)
    coord2d = np.stack([c0, c1], axis=1).reshape(FH, FW, 2).astype(np.float32)
    cbpad = np.pad(coord2d, ((1, 1), (1, 1), (0, 0)))

    valid = np.zeros((_FR,), np.float32)
    pfr = np.zeros((_FR, 24), np.float32)
    cba = np.zeros((_FR, 8), np.float32)
    for I in range(16):
        for J in range(16):
            p = 16 * I + J
            if 1 <= I <= FH and 1 <= J <= FW:
                valid[p] = 1.0
                cba[p, 0] = cbpad[I, J, 0]
                cba[p, 1] = cbpad[I, J, 1]
                cba[p, 2] = 1.0
                pfr[p, 18] = 1.0
                for di in range(3):
                    for dj in range(3):
                        for k in range(2):
                            pfr[p, (3 * di + dj) * 2 + k] = (
                                cbpad[I + di - 1, J + dj - 1, k])
    mask1k = np.ascontiguousarray(np.broadcast_to(
        np.tile(valid, _BLK // _FR)[:, None], (_BLK, 128))).astype(np.float32)
    return pfr, cba, mask1k


_PFR, _CBA, _MASK1K = _coord_consts()


def kernel(que, img, emb, Wih_f, Whh_f, bih_f, bhh_f, Wih_b, Whh_b, bih_b,
           bhh_b, Wc, bc, g0, bt0, W11, b11, W12, b12, g1, bt1, W21, b21,
           W22, b22, g2, bt2):
    f32 = jnp.float32

    table = jnp.concatenate([
        emb.astype(f32),
        jnp.ones((VOCAB + 1, 1), f32),
        jnp.zeros((VOCAB + 1, _KP - D_WORD - 1), f32)], axis=1)
    idx = jnp.concatenate([
        que.T.astype(jnp.int32).reshape(-1),
        jnp.zeros((_NTOK_PAD - _NTOK,), jnp.int32)])
    x = _sc_gather(table, idx)

    zpad = jnp.zeros((_KP - D_WORD - 1, 4 * D_HID), f32)
    wihf = jnp.concatenate([Wih_f.T, (bih_f + bhh_f)[None, :], zpad], axis=0)
    wihb = jnp.concatenate([Wih_b.T, (bih_b + bhh_b)[None, :], zpad], axis=0)
    enc = _lstm_call(x, wihf, Whh_f.T, wihb)

    vimg = jnp.transpose(img, (0, 2, 3, 1))
    vpad = jnp.pad(vimg, ((0, 0), (1, 1), (1, 1), (0, 0)))
    xp = vpad.reshape(_NP, 1024)

    wall = jnp.transpose(Wc[:, :1024], (1, 2, 3, 0)).reshape(1024, 9 * 128)
    wcc = jnp.stack([Wc[:, 1024 + k, di, dj]
                     for di in range(3) for dj in range(3) for k in range(2)],
                    axis=0)
    wcc = jnp.concatenate([wcc, bc[None, :], jnp.zeros((5, 128), f32)], axis=0)

    w11v = W11[:, :128, 0, 0].T
    w11c = jnp.concatenate([W11[:, 128, 0, 0][None], W11[:, 129, 0, 0][None],
                            b11[None], jnp.zeros((5, 128), f32)], axis=0)
    w12 = jnp.transpose(W12, (1, 2, 3, 0)).reshape(128, 9 * 128)
    w21v = W21[:, :128, 0, 0].T
    w21c = jnp.concatenate([W21[:, 128, 0, 0][None], W21[:, 129, 0, 0][None],
                            b21[None], jnp.zeros((5, 128), f32)], axis=0)
    w22 = jnp.transpose(W22, (1, 2, 3, 0)).reshape(128, 9 * 128)

    def rep8(v):
        return jnp.broadcast_to(v[None, :], (8, 128))

    pfr = jnp.asarray(_PFR)
    cba = jnp.asarray(_CBA)
    mask = jnp.asarray(_MASK1K)

    xn = _norm_call(xp)
    z1 = _proj_call(xn, wall, _BLK)
    yraw0, s0, s20 = _tap1_call(z1, pfr, wcc, mask)
    v11, raw1, s1, s21 = _res_front(
        yraw0, s0, s20, rep8(g0), rep8(bt0), cba, w11c, w11v, w12, mask)
    v12, raw2, s2_, s22 = _res_front_r(
        raw1, s1, s21, rep8(g1), rep8(bt1), cba, w21c, w21v, w22, mask,
        res=v11)
    vout = _final_call(raw2, s2_, s22, rep8(g2), rep8(bt2), mask, v12)

    v = jnp.transpose(
        vout.reshape(B, 16, 16, 128)[:, 1:15, 1:15, :], (0, 3, 1, 2))
    return enc, v
